# Initial kernel scaffold; baseline (speedup 1.0000x reference)
import jax
import jax.numpy as jnp
from jax.experimental import pallas as pl


def kernel(X, params):
    raise NotImplementedError("write your pallas kernel here")


# speedup vs baseline: 44.5693x; 44.5693x over previous
import jax
import jax.numpy as jnp
import numpy as np
from jax import lax
from jax.experimental import pallas as pl

ARITY_ = 4
NSUB = 16

def _mob_matrix():
    mob = np.zeros((NSUB, NSUB), dtype=np.float32)
    for t in range(NSUB):
        for s in range(NSUB):
            if s & ~t:
                continue
            mob[t, s] = (-1.0) ** (bin(t ^ s).count("1"))
    mob[0, :] = 0.0
    mob[0, 0] = 1.0
    return mob

_MOB = _mob_matrix()

BATCH_BLOCK = 512
GROUP_BLOCK = 8


def _hoa_body(mob_ref, x_ref, p_ref, o_ref):
    mob = mob_ref[...]
    for j in range(GROUP_BLOCK):
        cg = lax.dot_general(mob, p_ref[j], (((1,), (0,)), ((), ())),
                             preferred_element_type=jnp.float32,
                             precision=lax.Precision.HIGHEST)
        x = x_ref[j]
        a0, a1, a2, a3 = x[0:1], x[1:2], x[2:3], x[3:4]
        m3 = jnp.minimum(a0, a1)
        m5 = jnp.minimum(a0, a2)
        m6 = jnp.minimum(a1, a2)
        m9 = jnp.minimum(a0, a3)
        m10 = jnp.minimum(a1, a3)
        m12 = jnp.minimum(a2, a3)
        m7 = jnp.minimum(m3, a2)
        m11 = jnp.minimum(m3, a3)
        m13 = jnp.minimum(m5, a3)
        m14 = jnp.minimum(m6, a3)
        m15 = jnp.minimum(m7, a3)
        m0 = jnp.maximum(jnp.maximum(a0, a1), jnp.maximum(a2, a3))
        m = jnp.concatenate(
            [m0, a0, a1, m3, a2, m5, m6, m7, a3, m9, m10, m11, m12, m13, m14, m15],
            axis=0)
        out = lax.dot_general(m, cg, (((0,), (0,)), ((), ())),
                              preferred_element_type=jnp.float32,
                              precision=lax.Precision.HIGHEST)
        o_ref[:, j, :] = out


def kernel(X, params):
    B, G, A = X.shape
    D = params.shape[-1]
    XT = jnp.transpose(X, (1, 2, 0))
    grid = (G // GROUP_BLOCK, B // BATCH_BLOCK)
    return pl.pallas_call(
        _hoa_body,
        grid=grid,
        in_specs=[
            pl.BlockSpec((NSUB, NSUB), lambda g, b: (0, 0)),
            pl.BlockSpec((GROUP_BLOCK, A, BATCH_BLOCK), lambda g, b: (g, 0, b)),
            pl.BlockSpec((GROUP_BLOCK, NSUB, D), lambda g, b: (g, 0, 0)),
        ],
        out_specs=pl.BlockSpec((BATCH_BLOCK, GROUP_BLOCK, D), lambda g, b: (b, g, 0)),
        out_shape=jax.ShapeDtypeStruct((B, G, D), jnp.float32),
    )(jnp.asarray(_MOB), XT, params)


# speedup vs baseline: 107.5991x; 2.4142x over previous
import jax
import jax.numpy as jnp
import numpy as np
from jax import lax
from jax.experimental import pallas as pl

NSUB = 16
BATCH_BLOCK = 512
GROUP_BLOCK = 8


def _hoa_body(x_ref, c_ref, o_ref):
    a0 = x_ref[0]
    a1 = x_ref[1]
    a2 = x_ref[2]
    a3 = x_ref[3]
    m3 = jnp.minimum(a0, a1)
    m5 = jnp.minimum(a0, a2)
    m6 = jnp.minimum(a1, a2)
    m9 = jnp.minimum(a0, a3)
    m10 = jnp.minimum(a1, a3)
    m12 = jnp.minimum(a2, a3)
    m7 = jnp.minimum(m3, a2)
    m11 = jnp.minimum(m3, a3)
    m13 = jnp.minimum(m5, a3)
    m14 = jnp.minimum(m6, a3)
    m15 = jnp.minimum(m7, a3)
    m0 = jnp.maximum(jnp.maximum(a0, a1), jnp.maximum(a2, a3))
    m = jnp.concatenate(
        [m0, a0, a1, m3, a2, m5, m6, m7, a3, m9, m10, m11, m12, m13, m14, m15],
        axis=0)
    o_ref[...] = lax.dot_general(m, c_ref[0], (((0,), (0,)), ((), ())),
                                 preferred_element_type=jnp.float32,
                                 precision=lax.Precision.HIGHEST)


def kernel(X, params):
    B, G, A = X.shape
    D = params.shape[-1]
    GB, BB = GROUP_BLOCK, BATCH_BLOCK

    XT = jnp.transpose(X, (2, 1, 0))

    c = params.reshape(G, 2, 2, 2, 2, D)
    for ax in (1, 2, 3, 4):
        lo = lax.slice_in_dim(c, 0, 1, axis=ax)
        hi = lax.slice_in_dim(c, 1, 2, axis=ax)
        c = jnp.concatenate([lo, hi - lo], axis=ax)
    cmob = c.reshape(G, NSUB, D)
    cmob = cmob.at[:, 0, :].set(params[:, 0, :])
    cbd = jnp.einsum('cjtd,jk->ctjkd', cmob.reshape(G // GB, GB, NSUB, D),
                     jnp.eye(GB, dtype=cmob.dtype))
    cbd = cbd.reshape(G // GB, NSUB * GB, GB * D)

    grid = (G // GB, B // BB)
    out2 = pl.pallas_call(
        _hoa_body,
        grid=grid,
        in_specs=[
            pl.BlockSpec((A, GB, BB), lambda g, b: (0, g, b)),
            pl.BlockSpec((1, NSUB * GB, GB * D), lambda g, b: (g, 0, 0)),
        ],
        out_specs=pl.BlockSpec((BB, GB * D), lambda g, b: (b, g)),
        out_shape=jax.ShapeDtypeStruct((B, G * D), jnp.float32),
    )(XT, cbd)
    return out2.reshape(B, G, D)
